# baseline (speedup 1.0000x reference)
import jax
import jax.numpy as jnp
from jax.experimental import pallas as pl
from jax.experimental.pallas import tpu as pltpu

B, N, D = 1024, 77, 768
CBATCH = 16
NC = B // CBATCH
M = 8
L = 4
BIG = 2**30


def _body(pt_ref, tok_ref, emb_ref, pe_ref, out_ref, insem, outsem, buf, midx):
    def start_in(c):
        pltpu.make_async_copy(
            emb_ref.at[pl.ds(c * CBATCH, CBATCH)],
            buf.at[c % M], insem.at[c % M]).start()

    def wait_in(c):
        pltpu.make_async_copy(
            emb_ref.at[pl.ds(c * CBATCH, CBATCH)],
            buf.at[c % M], insem.at[c % M]).wait()

    def start_out(c):
        pltpu.make_async_copy(
            buf.at[c % M],
            out_ref.at[pl.ds(c * CBATCH, CBATCH)], outsem.at[c % M]).start()

    def wait_out(c):
        pltpu.make_async_copy(
            buf.at[c % M],
            out_ref.at[pl.ds(c * CBATCH, CBATCH)], outsem.at[c % M]).wait()

    tok = tok_ref[...]
    idx = (jax.lax.broadcasted_iota(jnp.int32, (B, N), 0) * N
           + jax.lax.broadcasted_iota(jnp.int32, (B, N), 1))
    midx[...] = jnp.where(tok == pt_ref[0], idx, BIG)

    for c in range(NC + L):
        if c < NC:
            if c >= M:
                wait_out(c - M)
            start_in(c)
        if c >= L:
            wait_in(c - L)
            start_out(c - L)
    for c in range(NC - M, NC):
        wait_out(c)

    first = jnp.min(midx[...])

    def overwrite(m):
        b = m // N
        n = m % N
        cp = pltpu.make_async_copy(
            pe_ref, out_ref.at[b, pl.ds(n, 1), :], insem.at[0])
        cp.start()
        cp.wait()
        midx[...] = jnp.where(idx == m, BIG, midx[...])
        return jnp.min(midx[...])

    jax.lax.while_loop(lambda m: m < BIG, overwrite, first)


def kernel(tokenized_text, embedded_text, placeholder_token, placeholder_embedding):
    pt = jnp.asarray(placeholder_token, jnp.int32).reshape(1)
    pe = placeholder_embedding.reshape(1, D)
    out = pl.pallas_call(
        _body,
        in_specs=[
            pl.BlockSpec(memory_space=pltpu.MemorySpace.SMEM),
            pl.BlockSpec(memory_space=pltpu.MemorySpace.VMEM),
            pl.BlockSpec(memory_space=pltpu.MemorySpace.HBM),
            pl.BlockSpec(memory_space=pltpu.MemorySpace.VMEM),
        ],
        out_specs=pl.BlockSpec(memory_space=pltpu.MemorySpace.HBM),
        out_shape=jax.ShapeDtypeStruct((B, N, D), jnp.float32),
        scratch_shapes=[
            pltpu.SemaphoreType.DMA((M,)),
            pltpu.SemaphoreType.DMA((M,)),
            pltpu.VMEM((M, CBATCH, N, D), jnp.float32),
            pltpu.VMEM((B, N), jnp.int32),
        ],
    )(pt, tokenized_text, embedded_text, pe)
    return out
